# Initial kernel scaffold; baseline (speedup 1.0000x reference)
import jax
import jax.numpy as jnp
from jax.experimental import pallas as pl


def kernel(x, W_qkv, W_dw, W_out, temperature, aw1, aw2, aw3, aw4):
    raise NotImplementedError("write your pallas kernel here")


# speedup vs baseline: 3.5398x; 3.5398x over previous
import functools

import jax
import jax.numpy as jnp
from jax import lax
from jax.experimental import pallas as pl
from jax.experimental.pallas import tpu as pltpu
from jax.experimental.pallas import tpu_sc as plsc

B, DIM, H, W = 2, 144, 224, 224
HEADS = 9
C = DIM // HEADS
OUT = 48
QKV = DIM * 3
TH = 16
NT = H // TH
HW_T = TH * W


def _k1_body(xm_ref, xt_ref, xb_ref, wqkv_ref, wdw_ref, temp_ref,
             v_ref, attn_ref, qkv_s, dw_s, ssq_s, ssk_s):
    i = pl.program_id(1)
    xm = xm_ref[0]
    xt = xt_ref[0, :, 7:8, :]
    xb = xb_ref[0, :, 0:1, :]
    zero_row = jnp.zeros_like(xt)
    xt = jnp.where(i == 0, zero_row, xt)
    xb = jnp.where(i == NT - 1, zero_row, xb)
    xfull = jnp.concatenate([xt, xm, xb], axis=1)
    qkv_s[...] = jnp.dot(
        wqkv_ref[...], xfull.reshape(DIM, (TH + 2) * W),
        preferred_element_type=jnp.float32).reshape(QKV, TH + 2, W)

    zcol = jnp.zeros((QKV, TH, 1), jnp.float32)

    def tap(idx):
        return wdw_ref[:, idx:idx + 1].reshape(QKV, 1, 1)

    def shifted(di, dj):
        rows = qkv_s[:, di:di + TH, :]
        if dj == 1:
            return rows
        if dj == 0:
            return jnp.concatenate([zcol, rows[:, :, :W - 1]], axis=2)
        return jnp.concatenate([rows[:, :, 1:], zcol], axis=2)

    dw_s[...] = tap(1) * shifted(0, 1)
    for di, dj in ((0, 0), (0, 2), (1, 0), (1, 1), (1, 2),
                   (2, 0), (2, 1), (2, 2)):
        dw_s[...] += tap(di * 3 + dj) * shifted(di, dj)

    v_ref[0] = dw_s[2 * DIM:]

    for h in range(HEADS):
        qh = dw_s[h * C:(h + 1) * C].reshape(C, HW_T)
        kh = dw_s[DIM + h * C:DIM + (h + 1) * C].reshape(C, HW_T)
        part = lax.dot_general(qh, kh, (((1,), (1,)), ((), ())),
                               preferred_element_type=jnp.float32)
        pssq = jnp.sum(qh * qh, axis=-1)
        pssk = jnp.sum(kh * kh, axis=-1)

        @pl.when(i == 0)
        def _():
            ssq_s[h, :] = pssq
            ssk_s[h, :] = pssk
            attn_ref[0, h] = part

        @pl.when(i > 0)
        def _():
            ssq_s[h, :] += pssq
            ssk_s[h, :] += pssk
            attn_ref[0, h] += part

    @pl.when(i == NT - 1)
    def _():
        for h in range(HEADS):
            nq = jnp.maximum(jnp.sqrt(ssq_s[h, :]), 1e-12)
            nk = jnp.maximum(jnp.sqrt(ssk_s[h, :]), 1e-12)
            th = temp_ref[h:h + 1, :]
            attn_ref[0, h] = (attn_ref[0, h]
                              / (nq[:, None] * nk[None, :]) * th)


def _k1_call(x, wqkv, wdw2, temp2):
    return pl.pallas_call(
        _k1_body,
        grid=(B, NT),
        in_specs=[
            pl.BlockSpec((1, DIM, TH, W), lambda b, i: (b, 0, i, 0)),
            pl.BlockSpec((1, DIM, 8, W),
                         lambda b, i: (b, 0, jnp.maximum(2 * i - 1, 0), 0)),
            pl.BlockSpec((1, DIM, 8, W),
                         lambda b, i: (b, 0, jnp.minimum(2 * i + 2, 2 * NT - 1), 0)),
            pl.BlockSpec((QKV, DIM), lambda b, i: (0, 0)),
            pl.BlockSpec((QKV, 9), lambda b, i: (0, 0)),
            pl.BlockSpec((HEADS, 1), lambda b, i: (0, 0)),
        ],
        out_specs=[
            pl.BlockSpec((1, DIM, TH, W), lambda b, i: (b, 0, i, 0)),
            pl.BlockSpec((1, HEADS, C, C), lambda b, i: (b, 0, 0, 0)),
        ],
        out_shape=[
            jax.ShapeDtypeStruct((B, DIM, H, W), jnp.float32),
            jax.ShapeDtypeStruct((B, HEADS, C, C), jnp.float32),
        ],
        scratch_shapes=[
            pltpu.VMEM((QKV, TH + 2, W), jnp.float32),
            pltpu.VMEM((QKV, TH, W), jnp.float32),
            pltpu.VMEM((HEADS, C), jnp.float32),
            pltpu.VMEM((HEADS, C), jnp.float32),
        ],
        compiler_params=pltpu.CompilerParams(
            dimension_semantics=("arbitrary", "arbitrary")),
    )(x, x, x, wqkv, wdw2, temp2)


N_MAT = B * HEADS


def _k3_body(attn_hbm, aws_hbm, out_hbm, cols_v, out_v, aws_v):
    wid = lax.axis_index("s") * 2 + lax.axis_index("c")

    @pl.when(wid < N_MAT)
    def _():
        pltpu.sync_copy(attn_hbm.at[wid], cols_v)
        pltpu.sync_copy(aws_hbm, aws_v)
        w1 = aws_v[0, :]
        w2 = aws_v[1, :]
        w3 = aws_v[2, :]
        zero = jnp.zeros((16,), jnp.float32)
        one = jnp.ones((16,), jnp.float32)
        cols = [cols_v[j, :] for j in range(16)]
        m = cols[0]
        for j in range(1, 16):
            m = jnp.maximum(m, cols[j])
        es = [jnp.exp(cols[j] - m) for j in range(16)]
        e8 = []
        e10 = []
        e12 = []
        s8 = zero
        s10 = zero
        s12 = zero
        for j in range(16):
            cj = cols[j]
            rank = zero
            for jp in range(16):
                if jp == j:
                    continue
                cjp = cols[jp]
                if jp < j:
                    beat = cjp >= cj
                else:
                    beat = cjp > cj
                rank = rank + jnp.where(beat, one, zero)
            ej = es[j]
            v8 = jnp.where(rank < 8.0, ej, zero)
            v10 = jnp.where(rank < 10.0, ej, zero)
            v12 = jnp.where(rank < 12.0, ej, zero)
            e8.append(v8)
            e10.append(v10)
            e12.append(v12)
            s8 = s8 + v8
            s10 = s10 + v10
            s12 = s12 + v12
        r8 = w1 / s8
        r10 = w2 / s10
        r12 = w3 / s12
        for j in range(16):
            out_v[j, :] = e8[j] * r8 + e10[j] * r10 + e12[j] * r12
        pltpu.sync_copy(out_v, out_hbm.at[wid])


def _k3_call(attn_t, aws3):
    mesh = plsc.VectorSubcoreMesh(core_axis_name="c", subcore_axis_name="s")
    fn = functools.partial(
        pl.kernel, mesh=mesh,
        out_type=jax.ShapeDtypeStruct((N_MAT, 16, 16), jnp.float32),
        scratch_types=[
            pltpu.VMEM((16, 16), jnp.float32),
            pltpu.VMEM((16, 16), jnp.float32),
            pltpu.VMEM((3, 16), jnp.float32),
        ],
    )(_k3_body)
    return fn(attn_t, aws3)


def _k4_body(a_ref, wout_ref, v_ref, out_ref, w2_s):
    i = pl.program_id(1)

    @pl.when(i == 0)
    def _():
        at = a_ref[0]
        for h in range(HEADS):
            wh = wout_ref[:, h * C:(h + 1) * C]
            w2_s[:, h * C:(h + 1) * C] = lax.dot_general(
                wh, at[h], (((1,), (1,)), ((), ())),
                preferred_element_type=jnp.float32)

    v2 = v_ref[0].reshape(DIM, HW_T)
    o = jnp.dot(w2_s[...], v2, preferred_element_type=jnp.float32)
    out_ref[0] = o.reshape(OUT, TH, W)


def _k4_call(a, wout, v):
    return pl.pallas_call(
        _k4_body,
        grid=(B, NT),
        in_specs=[
            pl.BlockSpec((1, HEADS, C, C), lambda b, i: (b, 0, 0, 0)),
            pl.BlockSpec((OUT, DIM), lambda b, i: (0, 0)),
            pl.BlockSpec((1, DIM, TH, W), lambda b, i: (b, 0, i, 0)),
        ],
        out_specs=pl.BlockSpec((1, OUT, TH, W), lambda b, i: (b, 0, i, 0)),
        out_shape=jax.ShapeDtypeStruct((B, OUT, H, W), jnp.float32),
        scratch_shapes=[pltpu.VMEM((OUT, DIM), jnp.float32)],
        compiler_params=pltpu.CompilerParams(
            dimension_semantics=("arbitrary", "arbitrary")),
    )(a, wout, v)


def kernel(x, W_qkv, W_dw, W_out, temperature, aw1, aw2, aw3, aw4):
    wdw2 = W_dw.reshape(QKV, 9)
    temp2 = temperature.reshape(HEADS, 1)
    v, attn = _k1_call(x, W_qkv, wdw2, temp2)
    aws3 = jnp.stack([
        jnp.broadcast_to(aw1, (16,)),
        jnp.broadcast_to(aw2, (16,)),
        jnp.broadcast_to(aw3 + aw4, (16,)),
    ]).astype(jnp.float32)
    attn_t = attn.transpose(0, 1, 3, 2).reshape(N_MAT, C, C)
    a_t = _k3_call(attn_t, aws3)
    return _k4_call(a_t.reshape(B, HEADS, C, C), W_out, v)
